# baseline (speedup 1.0000x reference)
import jax
import jax.numpy as jnp
from jax.experimental import pallas as pl
from jax.experimental.pallas import tpu as pltpu

N = 10000
F = 128
BM = 200


def _ginconv_body(eps_ref, xb_ref, adj_ref, xfull_ref, W1_ref, b1_ref,
                  W2_ref, b2_ref, out_ref):
    s = jax.lax.dot_general(
        adj_ref[...], xfull_ref[...],
        (((1,), (0,)), ((), ())), preferred_element_type=jnp.float32)
    s = s + (1.0 + eps_ref[0, 0]) * xb_ref[...]
    h = jax.lax.dot_general(
        s, W1_ref[...], (((1,), (1,)), ((), ())),
        preferred_element_type=jnp.float32)
    h = jnp.maximum(h + b1_ref[...], 0.0)
    o = jax.lax.dot_general(
        h, W2_ref[...], (((1,), (1,)), ((), ())),
        preferred_element_type=jnp.float32)
    out_ref[...] = o + b2_ref[...]


def kernel(x, adj, eps, W1, b1, W2, b2):
    eps2 = jnp.reshape(eps, (1, 1))
    b1_2 = jnp.reshape(b1, (1, F))
    b2_2 = jnp.reshape(b2, (1, F))
    grid = (N // BM,)
    out = pl.pallas_call(
        _ginconv_body,
        grid=grid,
        in_specs=[
            pl.BlockSpec((1, 1), lambda i: (0, 0), memory_space=pltpu.SMEM),
            pl.BlockSpec((BM, F), lambda i: (i, 0)),
            pl.BlockSpec((BM, N), lambda i: (i, 0)),
            pl.BlockSpec((N, F), lambda i: (0, 0)),
            pl.BlockSpec((F, F), lambda i: (0, 0)),
            pl.BlockSpec((1, F), lambda i: (0, 0)),
            pl.BlockSpec((F, F), lambda i: (0, 0)),
            pl.BlockSpec((1, F), lambda i: (0, 0)),
        ],
        out_specs=pl.BlockSpec((BM, F), lambda i: (i, 0)),
        out_shape=jax.ShapeDtypeStruct((N, F), jnp.float32),
        compiler_params=pltpu.CompilerParams(
            dimension_semantics=("arbitrary",),
            vmem_limit_bytes=100 * 1024 * 1024,
        ),
    )(eps2, x, adj, x, W1, b1_2, W2, b2_2)
    return out


# speedup vs baseline: 1.0256x; 1.0256x over previous
import jax
import jax.numpy as jnp
from jax.experimental import pallas as pl
from jax.experimental.pallas import tpu as pltpu

N = 10000
F = 128
BM = 512


def _ginconv_body(eps_ref, xb_ref, adj_ref, xfull_ref, W1_ref, b1_ref,
                  W2_ref, b2_ref, out_ref):
    s = jax.lax.dot_general(
        adj_ref[...], xfull_ref[...],
        (((1,), (0,)), ((), ())), preferred_element_type=jnp.float32)
    s = s + (1.0 + eps_ref[0, 0]) * xb_ref[...]
    h = jax.lax.dot_general(
        s, W1_ref[...], (((1,), (1,)), ((), ())),
        preferred_element_type=jnp.float32)
    h = jnp.maximum(h + b1_ref[...], 0.0)
    o = jax.lax.dot_general(
        h, W2_ref[...], (((1,), (1,)), ((), ())),
        preferred_element_type=jnp.float32)
    out_ref[...] = o + b2_ref[...]


def kernel(x, adj, eps, W1, b1, W2, b2):
    eps2 = jnp.reshape(eps, (1, 1))
    b1_2 = jnp.reshape(b1, (1, F))
    b2_2 = jnp.reshape(b2, (1, F))
    grid = (pl.cdiv(N, BM),)
    out = pl.pallas_call(
        _ginconv_body,
        grid=grid,
        in_specs=[
            pl.BlockSpec((1, 1), lambda i: (0, 0), memory_space=pltpu.SMEM),
            pl.BlockSpec((BM, F), lambda i: (i, 0)),
            pl.BlockSpec((BM, N), lambda i: (i, 0)),
            pl.BlockSpec((N, F), lambda i: (0, 0)),
            pl.BlockSpec((F, F), lambda i: (0, 0)),
            pl.BlockSpec((1, F), lambda i: (0, 0)),
            pl.BlockSpec((F, F), lambda i: (0, 0)),
            pl.BlockSpec((1, F), lambda i: (0, 0)),
        ],
        out_specs=pl.BlockSpec((BM, F), lambda i: (i, 0)),
        out_shape=jax.ShapeDtypeStruct((N, F), jnp.float32),
        compiler_params=pltpu.CompilerParams(
            dimension_semantics=("arbitrary",),
            vmem_limit_bytes=100 * 1024 * 1024,
        ),
    )(eps2, x, adj, x, W1, b1_2, W2, b2_2)
    return out


# speedup vs baseline: 1.0396x; 1.0136x over previous
import jax
import jax.numpy as jnp
from jax.experimental import pallas as pl
from jax.experimental.pallas import tpu as pltpu

N = 10000
F = 128
BM = 400


def _ginconv_body(eps_ref, xb_ref, adj_ref, xfull_ref, W1_ref, b1_ref,
                  W2_ref, b2_ref, out_ref):
    s = jax.lax.dot_general(
        adj_ref[...], xfull_ref[...],
        (((1,), (0,)), ((), ())), preferred_element_type=jnp.float32)
    s = s + (1.0 + eps_ref[0, 0]) * xb_ref[...]
    h = jax.lax.dot_general(
        s, W1_ref[...], (((1,), (1,)), ((), ())),
        preferred_element_type=jnp.float32)
    h = jnp.maximum(h + b1_ref[...], 0.0)
    o = jax.lax.dot_general(
        h, W2_ref[...], (((1,), (1,)), ((), ())),
        preferred_element_type=jnp.float32)
    out_ref[...] = o + b2_ref[...]


def kernel(x, adj, eps, W1, b1, W2, b2):
    eps2 = jnp.reshape(eps, (1, 1))
    b1_2 = jnp.reshape(b1, (1, F))
    b2_2 = jnp.reshape(b2, (1, F))
    grid = (pl.cdiv(N, BM),)
    out = pl.pallas_call(
        _ginconv_body,
        grid=grid,
        in_specs=[
            pl.BlockSpec((1, 1), lambda i: (0, 0), memory_space=pltpu.SMEM),
            pl.BlockSpec((BM, F), lambda i: (i, 0)),
            pl.BlockSpec((BM, N), lambda i: (i, 0)),
            pl.BlockSpec((N, F), lambda i: (0, 0)),
            pl.BlockSpec((F, F), lambda i: (0, 0)),
            pl.BlockSpec((1, F), lambda i: (0, 0)),
            pl.BlockSpec((F, F), lambda i: (0, 0)),
            pl.BlockSpec((1, F), lambda i: (0, 0)),
        ],
        out_specs=pl.BlockSpec((BM, F), lambda i: (i, 0)),
        out_shape=jax.ShapeDtypeStruct((N, F), jnp.float32),
        compiler_params=pltpu.CompilerParams(
            dimension_semantics=("arbitrary",),
            vmem_limit_bytes=100 * 1024 * 1024,
        ),
    )(eps2, x, adj, x, W1, b1_2, W2, b2_2)
    return out


# speedup vs baseline: 1.0566x; 1.0164x over previous
import jax
import jax.numpy as jnp
from jax.experimental import pallas as pl
from jax.experimental.pallas import tpu as pltpu

N = 10000
F = 128
BM = 400


def _ginconv_body(eps_ref, adj_ref, xfull_ref, W1_ref, b1_ref,
                  W2_ref, b2_ref, out_ref):
    s = jax.lax.dot_general(
        adj_ref[...], xfull_ref[...],
        (((1,), (0,)), ((), ())), preferred_element_type=jnp.float32)
    i = pl.program_id(0)
    xb = xfull_ref[pl.ds(i * BM, BM), :]
    s = s + (1.0 + eps_ref[0, 0]) * xb
    h = jax.lax.dot_general(
        s, W1_ref[...], (((1,), (1,)), ((), ())),
        preferred_element_type=jnp.float32)
    h = jnp.maximum(h + b1_ref[...], 0.0)
    o = jax.lax.dot_general(
        h, W2_ref[...], (((1,), (1,)), ((), ())),
        preferred_element_type=jnp.float32)
    out_ref[...] = o + b2_ref[...]


def kernel(x, adj, eps, W1, b1, W2, b2):
    eps2 = jnp.reshape(eps, (1, 1))
    b1_2 = jnp.reshape(b1, (1, F))
    b2_2 = jnp.reshape(b2, (1, F))
    grid = (pl.cdiv(N, BM),)
    out = pl.pallas_call(
        _ginconv_body,
        grid=grid,
        in_specs=[
            pl.BlockSpec((1, 1), lambda i: (0, 0), memory_space=pltpu.SMEM),
            pl.BlockSpec((BM, N), lambda i: (i, 0)),
            pl.BlockSpec((N, F), lambda i: (0, 0)),
            pl.BlockSpec((F, F), lambda i: (0, 0)),
            pl.BlockSpec((1, F), lambda i: (0, 0)),
            pl.BlockSpec((F, F), lambda i: (0, 0)),
            pl.BlockSpec((1, F), lambda i: (0, 0)),
        ],
        out_specs=pl.BlockSpec((BM, F), lambda i: (i, 0)),
        out_shape=jax.ShapeDtypeStruct((N, F), jnp.float32),
        compiler_params=pltpu.CompilerParams(
            dimension_semantics=("arbitrary",),
            vmem_limit_bytes=100 * 1024 * 1024,
        ),
    )(eps2, adj, x, W1, b1_2, W2, b2_2)
    return out
